# baseline (speedup 1.0000x reference)
import functools

import jax
import jax.numpy as jnp
from jax import lax
from jax.experimental import pallas as pl
from jax.experimental.pallas import tpu as pltpu
from jax.experimental.pallas import tpu_sc as plsc

NC = 2
NS = 16
L = 16
NW = NC * NS


@functools.partial(jax.jit, static_argnums=(3, 4, 5))
def _sc_segsum(src_flat, dst_flat, table, n_pad, rows_per_tile, with_counts):
    d = table.shape[1]
    bsz = 128
    n_steps = rows_per_tile * 128 // bsz
    zrows = n_pad // NS
    nz = zrows // bsz
    zrem = zrows % bsz

    mesh = plsc.VectorSubcoreMesh(
        core_axis_name="c", subcore_axis_name="s", num_cores=NC, num_subcores=NS)

    S = 3
    out_type = [jax.ShapeDtypeStruct((NC, n_pad, d), jnp.float32)]
    scratch = (
        [pltpu.VMEM((bsz,), jnp.int32) for _ in range(2 * S)]
        + [pltpu.VMEM((bsz, d), jnp.float32) for _ in range(S)]
        + [pltpu.VMEM_SHARED((n_pad, d), jnp.float32)]
        + [pltpu.SemaphoreType.DMA for _ in range(2 * S)]
    )
    if with_counts:
        out_type.append(jax.ShapeDtypeStruct((NC, n_pad, d), jnp.float32))

    def body(src_hbm, dst_hbm, tab_hbm, agg_out, *rest):
        if with_counts:
            cnt_out = rest[0]
            rest = rest[1:]
        idx_refs = rest[:2 * S]
        row_refs = rest[2 * S:3 * S]
        agg_sh = rest[3 * S]
        sem_g = rest[3 * S + 1:3 * S + 1 + S]
        sem_s = rest[3 * S + 1 + S:3 * S + 1 + 2 * S]
        slots = tuple((idx_refs[2 * b], idx_refs[2 * b + 1], row_refs[b],
                       sem_g[b], sem_s[b]) for b in range(S))
        rows_v0 = row_refs[0]
        dst_v0, dst_v1 = idx_refs[1], idx_refs[3]
        sem0, sem1 = sem_s[0], sem_s[1]
        cid = lax.axis_index("c")
        sid = lax.axis_index("s")
        wid = sid * NC + cid

        def zr(i, carry):
            for k in range(d // L):
                rows_v0[i, pl.ds(k * L, L)] = jnp.zeros((L,), jnp.float32)
            return carry
        lax.fori_loop(0, bsz, zr, 0)

        base = sid * zrows
        def zs(k, carry):
            pltpu.sync_copy(rows_v0, agg_sh.at[pl.ds(base + k * bsz, bsz)])
            return carry
        lax.fori_loop(0, nz, zs, 0)
        if zrem:
            pltpu.sync_copy(rows_v0.at[pl.ds(0, zrem)],
                            agg_sh.at[pl.ds(base + nz * bsz, zrem)])

        def _off(j):
            return pl.multiple_of(wid * rows_per_tile * 128 + j * bsz, bsz)

        def _stage_fire(j, b):
            s_v, d_v, r_v, smg, _ = slots[b]
            off = _off(j)
            pltpu.sync_copy(src_hbm.at[pl.ds(off, bsz)], s_v)
            pltpu.sync_copy(dst_hbm.at[pl.ds(off, bsz)], d_v)
            pltpu.async_copy(tab_hbm.at[s_v], r_v, smg)

        plsc.subcore_barrier()

        for b in range(S):
            _stage_fire(b, b)

        def group(g, carry):
            for t in range(S):
                j = g * S + t
                s_v, d_v, r_v, smg, sms = slots[t]
                pltpu.make_async_copy(tab_hbm.at[s_v], r_v, smg).wait()
                pltpu.async_copy(r_v, agg_sh.at[d_v], sms, add=True)

                m = j - (S - 1)
                b2 = (t + 1) % S
                s2, d2, r2, smg2, sms2 = slots[b2]

                @pl.when(m >= 0)
                def _():
                    pltpu.make_async_copy(r2, agg_sh.at[d2], sms2).wait()

                    @pl.when(m + S < n_steps)
                    def _():
                        _stage_fire(m + S, b2)
            return carry
        lax.fori_loop(0, n_steps // S, group, 0)

        for j in range(n_steps - (S - 1), n_steps):
            _, d_v, r_v, _, sms = slots[j % S]
            pltpu.make_async_copy(r_v, agg_sh.at[d_v], sms).wait()

        plsc.subcore_barrier()

        pltpu.sync_copy(agg_sh.at[pl.ds(base, zrows)],
                        agg_out.at[cid, pl.ds(base, zrows)])

        if with_counts:
            lax.fori_loop(0, bsz, zr, 0)
            lax.fori_loop(0, nz, zs, 0)
            if zrem:
                pltpu.sync_copy(rows_v0.at[pl.ds(0, zrem)],
                                agg_sh.at[pl.ds(base + nz * bsz, zrem)])

            def zo(i, carry):
                for k in range(d // L):
                    rows_v0[i, pl.ds(k * L, L)] = jnp.ones((L,), jnp.float32)
                return carry
            lax.fori_loop(0, bsz, zo, 0)

            pltpu.sync_copy(dst_hbm.at[pl.ds(_off(0), bsz)], dst_v0)
            pltpu.sync_copy(dst_hbm.at[pl.ds(_off(1), bsz)], dst_v1)
            plsc.subcore_barrier()

            pltpu.async_copy(rows_v0, agg_sh.at[dst_v0], sem0, add=True)

            def pair2(g, carry):
                for b in range(2):
                    j = 2 * g + b
                    d_v = (dst_v0, dst_v1)[b]
                    d_n = (dst_v0, dst_v1)[1 - b]
                    sm = (sem0, sem1)[b]
                    sm_n = (sem0, sem1)[1 - b]

                    @pl.when(j < n_steps)
                    def _():
                        @pl.when(j + 1 < n_steps)
                        def _():
                            pltpu.async_copy(rows_v0, agg_sh.at[d_n], sm_n,
                                             add=True)
                        pltpu.make_async_copy(rows_v0, agg_sh.at[d_v],
                                              sm).wait()

                        @pl.when(j + 2 < n_steps)
                        def _():
                            pltpu.sync_copy(
                                dst_hbm.at[pl.ds(_off(j + 2), bsz)], d_v)
                return carry
            lax.fori_loop(0, (n_steps + 1) // 2, pair2, 0)
            plsc.subcore_barrier()

            pltpu.sync_copy(agg_sh.at[pl.ds(base, zrows)],
                            cnt_out.at[cid, pl.ds(base, zrows)])

    k = pl.kernel(body, out_type=out_type, mesh=mesh, scratch_types=scratch)
    return k(src_flat, dst_flat, table)


def _dgT(x, w):
    return lax.dot_general(x, w, (((1,), (1,)), ((), ())),
                           preferred_element_type=jnp.float32)


def _mm_pair(x, wa, wb, b, blk=2000):
    n, d = x.shape
    h = wa.shape[0]

    def body(x_ref, wa_ref, wb_ref, b_ref, o1_ref, o2_ref):
        xb = x_ref[...]
        o1_ref[...] = _dgT(xb, wa_ref[...])
        o2_ref[...] = _dgT(xb, wb_ref[...]) + b_ref[...]

    return pl.pallas_call(
        body,
        grid=(n // blk,),
        in_specs=[pl.BlockSpec((blk, d), lambda i: (i, 0)),
                  pl.BlockSpec((h, d), lambda i: (0, 0)),
                  pl.BlockSpec((h, d), lambda i: (0, 0)),
                  pl.BlockSpec((1, h), lambda i: (0, 0))],
        out_specs=[pl.BlockSpec((blk, h), lambda i: (i, 0)),
                   pl.BlockSpec((blk, h), lambda i: (i, 0))],
        out_shape=[jax.ShapeDtypeStruct((n, h), jnp.float32),
                   jax.ShapeDtypeStruct((n, h), jnp.float32)],
    )(x, wa, wb, b.reshape(1, h))


def _combine(aggp, cnt, xr, ws, b, blk=2000):
    n, h = xr.shape

    def body(agg_ref, cnt_ref, xr_ref, *rest):
        w_refs = rest[:len(ws)]
        b_ref = rest[len(ws)]
        o_refs = rest[len(ws) + 1:]
        feats = agg_ref[0] + agg_ref[1]
        c = cnt_ref[0, :, 0:1] + cnt_ref[1, :, 0:1]
        mean = feats / jnp.maximum(c, 1.0)
        hb = jnp.maximum(mean + xr_ref[...], 0.0)
        for i, (w_ref, o_ref) in enumerate(zip(w_refs, o_refs)):
            r = _dgT(hb, w_ref[...])
            if i == len(ws) - 1:
                r = r + b_ref[...]
            o_ref[...] = r

    in_specs = [pl.BlockSpec((NC, blk, h), lambda i: (0, i, 0)),
                pl.BlockSpec((NC, blk, h), lambda i: (0, i, 0)),
                pl.BlockSpec((blk, h), lambda i: (i, 0))]
    in_specs += [pl.BlockSpec((w.shape[0], h), lambda i: (0, 0)) for w in ws]
    in_specs.append(pl.BlockSpec((1, ws[-1].shape[0]), lambda i: (0, 0)))

    return pl.pallas_call(
        body,
        grid=(n // blk,),
        in_specs=in_specs,
        out_specs=[pl.BlockSpec((blk, w.shape[0]), lambda i: (i, 0)) for w in ws],
        out_shape=[jax.ShapeDtypeStruct((n, w.shape[0]), jnp.float32) for w in ws],
    )(aggp, cnt, xr, *ws, b.reshape(1, -1))


def kernel(x, edge_index, W1l, W1r, b1, W2l, W2r, b2, Wc, bc):
    n, d = x.shape
    e = edge_index.shape[1]

    rows_per_tile = -(-(-(-e // (NW * 128))) // 3) * 3
    e_pad = rows_per_tile * NW * 128
    n_pad = -(-(n + 1) // (NS * 8)) * (NS * 8)

    src = edge_index[0]
    dst = edge_index[1]
    pad = e_pad - e
    src_flat = jnp.concatenate([src, jnp.zeros((pad,), jnp.int32)])
    dst_flat = jnp.concatenate([dst, jnp.full((pad,), n, jnp.int32)])

    xl1, xr1 = _mm_pair(x, W1l, W1r, b1)
    agg1, cnt = _sc_segsum(src_flat, dst_flat, xl1, n_pad, rows_per_tile, True)
    xl2, xr2 = _combine(agg1, cnt, xr1, [W2l, W2r], b2)
    agg2 = _sc_segsum(src_flat, dst_flat, xl2, n_pad, rows_per_tile, False)[0]
    out = _combine(agg2, cnt, xr2, [Wc], bc)[0]
    return out


# speedup vs baseline: 1.7579x; 1.7579x over previous
import functools

import jax
import jax.numpy as jnp
from jax import lax
from jax.experimental import pallas as pl
from jax.experimental.pallas import tpu as pltpu
from jax.experimental.pallas import tpu_sc as plsc

NC = 2
NS = 16
L = 16
NW = NC * NS


@functools.partial(jax.jit, static_argnums=(3, 4, 5, 6))
def _sc_segsum(src_flat, dst_flat, table, n_pad, blocks_c0, blocks_c1,
               with_counts):
    d = table.shape[1]
    bsz = 128
    zrows = n_pad // NS
    nz = zrows // bsz
    zrem = zrows % bsz

    mesh = plsc.VectorSubcoreMesh(
        core_axis_name="c", subcore_axis_name="s", num_cores=NC, num_subcores=NS)

    out_type = [jax.ShapeDtypeStruct((NC, n_pad, d), jnp.float32)]
    scratch = [
        pltpu.VMEM((bsz,), jnp.int32),
        pltpu.VMEM((bsz,), jnp.int32),
        pltpu.VMEM((bsz,), jnp.int32),
        pltpu.VMEM((bsz,), jnp.int32),
        pltpu.VMEM((bsz, d), jnp.float32),
        pltpu.VMEM((bsz, d), jnp.float32),
        pltpu.VMEM_SHARED((n_pad, d), jnp.float32),
        pltpu.SemaphoreType.DMA,
        pltpu.SemaphoreType.DMA,
    ]
    if with_counts:
        out_type.append(jax.ShapeDtypeStruct((NC, n_pad, d), jnp.float32))

    def body(src_hbm, dst_hbm, tab_hbm, agg_out, *rest):
        if with_counts:
            (cnt_out, src_v0, dst_v0, src_v1, dst_v1, rows_v0, rows_v1,
             agg_sh, sem0, sem1) = rest
        else:
            (src_v0, dst_v0, src_v1, dst_v1, rows_v0, rows_v1,
             agg_sh, sem0, sem1) = rest
        cid = lax.axis_index("c")
        sid = lax.axis_index("s")
        bufs = ((src_v0, dst_v0, rows_v0, sem0),
                (src_v1, dst_v1, rows_v1, sem1))
        base_blk = jnp.where(cid == 0, sid * blocks_c0,
                             NS * blocks_c0 + sid * blocks_c1)
        n_steps = jnp.where(cid == 0, blocks_c0, blocks_c1)

        def zr(i, carry):
            for k in range(d // L):
                rows_v0[i, pl.ds(k * L, L)] = jnp.zeros((L,), jnp.float32)
            return carry
        lax.fori_loop(0, bsz, zr, 0)

        base = sid * zrows
        def zs(k, carry):
            pltpu.sync_copy(rows_v0, agg_sh.at[pl.ds(base + k * bsz, bsz)])
            return carry
        lax.fori_loop(0, nz, zs, 0)
        if zrem:
            pltpu.sync_copy(rows_v0.at[pl.ds(0, zrem)],
                            agg_sh.at[pl.ds(base + nz * bsz, zrem)])

        def _off(j):
            return pl.multiple_of((base_blk + j) * bsz, bsz)

        def _stage_fire(j, b):
            s_v, d_v, r_v, sm = bufs[b]
            off = _off(j)
            pltpu.sync_copy(src_hbm.at[pl.ds(off, bsz)], s_v)
            pltpu.sync_copy(dst_hbm.at[pl.ds(off, bsz)], d_v)
            pltpu.async_copy(tab_hbm.at[s_v], r_v, sm)

        plsc.subcore_barrier()

        _stage_fire(0, 0)
        _stage_fire(1, 1)

        def pair(g, carry):
            for b in range(2):
                j = 2 * g + b
                s_v, d_v, r_v, sm = bufs[b]
                pltpu.make_async_copy(tab_hbm.at[s_v], r_v, sm).wait()
                pltpu.sync_copy(r_v, agg_sh.at[d_v], add=True)

                @pl.when(j + 2 < n_steps)
                def _():
                    _stage_fire(j + 2, b)
            return carry
        lax.fori_loop(0, n_steps // 2, pair, 0)

        plsc.subcore_barrier()

        pltpu.sync_copy(agg_sh.at[pl.ds(base, zrows)],
                        agg_out.at[cid, pl.ds(base, zrows)])

        if with_counts:
            lax.fori_loop(0, bsz, zr, 0)
            lax.fori_loop(0, nz, zs, 0)
            if zrem:
                pltpu.sync_copy(rows_v0.at[pl.ds(0, zrem)],
                                agg_sh.at[pl.ds(base + nz * bsz, zrem)])

            def zo(i, carry):
                for k in range(d // L):
                    rows_v0[i, pl.ds(k * L, L)] = jnp.ones((L,), jnp.float32)
                return carry
            lax.fori_loop(0, bsz, zo, 0)

            pltpu.sync_copy(dst_hbm.at[pl.ds(_off(0), bsz)], dst_v0)
            pltpu.sync_copy(dst_hbm.at[pl.ds(_off(1), bsz)], dst_v1)
            plsc.subcore_barrier()

            pltpu.async_copy(rows_v0, agg_sh.at[dst_v0], sem0, add=True)

            def pair2(g, carry):
                for b in range(2):
                    j = 2 * g + b
                    d_v = (dst_v0, dst_v1)[b]
                    d_n = (dst_v0, dst_v1)[1 - b]
                    sm = (sem0, sem1)[b]
                    sm_n = (sem0, sem1)[1 - b]

                    @pl.when(j < n_steps)
                    def _():
                        @pl.when(j + 1 < n_steps)
                        def _():
                            pltpu.async_copy(rows_v0, agg_sh.at[d_n], sm_n,
                                             add=True)
                        pltpu.make_async_copy(rows_v0, agg_sh.at[d_v],
                                              sm).wait()

                        @pl.when(j + 2 < n_steps)
                        def _():
                            pltpu.sync_copy(
                                dst_hbm.at[pl.ds(_off(j + 2), bsz)], d_v)
                return carry
            lax.fori_loop(0, (n_steps + 1) // 2, pair2, 0)
            plsc.subcore_barrier()

            pltpu.sync_copy(agg_sh.at[pl.ds(base, zrows)],
                            cnt_out.at[cid, pl.ds(base, zrows)])

    k = pl.kernel(body, out_type=out_type, mesh=mesh, scratch_types=scratch)
    return k(src_flat, dst_flat, table)


def _dgT(x, w):
    return lax.dot_general(x, w, (((1,), (1,)), ((), ())),
                           preferred_element_type=jnp.float32)


def _mm_pair(x, wa, wb, b, blk=2000):
    n, d = x.shape
    h = wa.shape[0]

    def body(x_ref, wa_ref, wb_ref, b_ref, o1_ref, o2_ref):
        xb = x_ref[...]
        o1_ref[...] = _dgT(xb, wa_ref[...])
        o2_ref[...] = _dgT(xb, wb_ref[...]) + b_ref[...]

    return pl.pallas_call(
        body,
        grid=(n // blk,),
        in_specs=[pl.BlockSpec((blk, d), lambda i: (i, 0)),
                  pl.BlockSpec((h, d), lambda i: (0, 0)),
                  pl.BlockSpec((h, d), lambda i: (0, 0)),
                  pl.BlockSpec((1, h), lambda i: (0, 0))],
        out_specs=[pl.BlockSpec((blk, h), lambda i: (i, 0)),
                   pl.BlockSpec((blk, h), lambda i: (i, 0))],
        out_shape=[jax.ShapeDtypeStruct((n, h), jnp.float32),
                   jax.ShapeDtypeStruct((n, h), jnp.float32)],
    )(x, wa, wb, b.reshape(1, h))


def _combine(aggp, cnt, xr, ws, b, blk=2000):
    n, h = xr.shape

    def body(agg_ref, cnt_ref, xr_ref, *rest):
        w_refs = rest[:len(ws)]
        b_ref = rest[len(ws)]
        o_refs = rest[len(ws) + 1:]
        feats = agg_ref[0] + agg_ref[1]
        c = cnt_ref[0, :, 0:1] + cnt_ref[1, :, 0:1]
        mean = feats / jnp.maximum(c, 1.0)
        hb = jnp.maximum(mean + xr_ref[...], 0.0)
        for i, (w_ref, o_ref) in enumerate(zip(w_refs, o_refs)):
            r = _dgT(hb, w_ref[...])
            if i == len(ws) - 1:
                r = r + b_ref[...]
            o_ref[...] = r

    in_specs = [pl.BlockSpec((NC, blk, h), lambda i: (0, i, 0)),
                pl.BlockSpec((NC, blk, h), lambda i: (0, i, 0)),
                pl.BlockSpec((blk, h), lambda i: (i, 0))]
    in_specs += [pl.BlockSpec((w.shape[0], h), lambda i: (0, 0)) for w in ws]
    in_specs.append(pl.BlockSpec((1, ws[-1].shape[0]), lambda i: (0, 0)))

    return pl.pallas_call(
        body,
        grid=(n // blk,),
        in_specs=in_specs,
        out_specs=[pl.BlockSpec((blk, w.shape[0]), lambda i: (i, 0)) for w in ws],
        out_shape=[jax.ShapeDtypeStruct((n, w.shape[0]), jnp.float32) for w in ws],
    )(aggp, cnt, xr, *ws, b.reshape(1, -1))


def kernel(x, edge_index, W1l, W1r, b1, W2l, W2r, b2, Wc, bc):
    n, d = x.shape
    e = edge_index.shape[1]

    blocks = -(-e // 128)
    pair_blocks = -(-blocks // NS)
    pair_blocks += pair_blocks % 2

    bc0 = max(2, (pair_blocks // 3) & ~1)
    e_pad = NS * pair_blocks * 128
    n_pad = -(-(n + 1) // (NS * 8)) * (NS * 8)

    src = edge_index[0]
    dst = edge_index[1]
    pad = e_pad - e
    src_flat = jnp.concatenate([src, jnp.zeros((pad,), jnp.int32)])
    dst_flat = jnp.concatenate([dst, jnp.full((pad,), n, jnp.int32)])

    xl1, xr1 = _mm_pair(x, W1l, W1r, b1)
    agg1, cnt = _sc_segsum(src_flat, dst_flat, xl1, n_pad, bc0,
                           pair_blocks - bc0, True)
    xl2, xr2 = _combine(agg1, cnt, xr1, [W2l, W2r], b2)
    agg2 = _sc_segsum(src_flat, dst_flat, xl2, n_pad, bc0,
                      pair_blocks - bc0, False)[0]
    out = _combine(agg2, cnt, xr2, [Wc], bc)[0]
    return out
